# baseline (speedup 1.0000x reference)
import functools

import jax
import jax.numpy as jnp
from jax import lax
from jax.experimental import pallas as pl
from jax.experimental.pallas import tpu as pltpu
from jax.experimental.pallas import tpu_sc as plsc

N_NODES = 10000
N_EDGES = 320000
D = 128

NC = 2
NS = 16
HALF_NODES = N_NODES // NC
CH = 128
CHUNKS_PER_TILE = 157
E_PAD = NS * CHUNKS_PER_TILE * CH
ROWS_PER_TILE = 312
TAIL_ROWS = HALF_NODES - NS * ROWS_PER_TILE


def _matmul_body(x_ref, w_ref, o_ref):
    o_ref[...] = jnp.dot(x_ref[...], w_ref[...],
                         preferred_element_type=jnp.float32)


def _matmul(x, W):
    blk = 2000
    return pl.pallas_call(
        _matmul_body,
        grid=(N_NODES // blk,),
        in_specs=[
            pl.BlockSpec((blk, D), lambda i: (i, 0)),
            pl.BlockSpec((D, D), lambda i: (0, 0)),
        ],
        out_specs=pl.BlockSpec((blk, D), lambda i: (i, 0)),
        out_shape=jax.ShapeDtypeStruct((N_NODES, D), jnp.float32),
    )(x, W)


def _spmm_body(support_hbm, col_hbm, row_hbm, ew_hbm, b_hbm, out_hbm,
               acc_shared, colbuf, rowidx, wbuf, rowsbuf, bbuf, zbuf, sem,
               sem_s):
    c = lax.axis_index("c")
    s = lax.axis_index("s")
    lo = c * HALF_NODES

    pltpu.sync_copy(b_hbm, bbuf)

    def _fill_body(r, _):
        for j in range(D // 16):
            sl = pl.ds(j * 16, 16)
            zbuf[r, sl] = bbuf[sl]
        return 0
    lax.fori_loop(0, ROWS_PER_TILE, _fill_body, 0)
    pltpu.sync_copy(zbuf, acc_shared.at[pl.ds(s * ROWS_PER_TILE, ROWS_PER_TILE)])

    @pl.when(s == NS - 1)
    def _fill_tail():
        pltpu.sync_copy(zbuf.at[pl.ds(0, TAIL_ROWS)],
                        acc_shared.at[pl.ds(NS * ROWS_PER_TILE, TAIL_ROWS)])
    plsc.subcore_barrier()

    zv = jnp.zeros((16,), jnp.float32)

    def _zrb_body(e, _):
        for j in range(D // 16):
            rowsbuf[e, pl.ds(j * 16, 16)] = zv
        return 0
    lax.fori_loop(0, CH, _zrb_body, 0)
    zidx = jnp.zeros((16,), jnp.int32)
    for g in range(CH // 16):
        rowidx[pl.ds(g * 16, 16)] = zidx
    pltpu.async_copy(rowsbuf, acc_shared.at[rowidx], sem_s, add=True)

    def _chunk_body(k, _):
        base = (s * CHUNKS_PER_TILE + k) * CH
        pltpu.sync_copy(col_hbm.at[pl.ds(base, CH)], colbuf)
        pltpu.sync_copy(ew_hbm.at[pl.ds(base, CH)], wbuf)
        pltpu.make_async_copy(rowsbuf, acc_shared.at[rowidx], sem_s).wait()
        gather = pltpu.async_copy(support_hbm.at[colbuf], rowsbuf, sem)
        pltpu.sync_copy(row_hbm.at[pl.ds(base, CH)], rowidx)
        gather.wait()

        def _scale_body(g, _):
            sl16 = pl.ds(g * 16, 16)
            rl = rowidx[sl16] - lo
            m = (rl >= 0) & (rl < HALF_NODES)
            wsel = jnp.where(m, wbuf[sl16], 0.0)
            rowidx[sl16] = jnp.where(m, rl, 0)
            for e2 in range(16):
                w = wsel[e2]
                e = g * 16 + e2
                for j in range(D // 16):
                    sl = pl.ds(j * 16, 16)
                    rowsbuf[e, sl] = rowsbuf[e, sl] * w
            return 0
        lax.fori_loop(0, CH // 16, _scale_body, 0)

        pltpu.async_copy(rowsbuf, acc_shared.at[rowidx], sem_s, add=True)
        return 0
    lax.fori_loop(0, CHUNKS_PER_TILE, _chunk_body, 0)
    pltpu.make_async_copy(rowsbuf, acc_shared.at[rowidx], sem_s).wait()

    plsc.subcore_barrier()
    pltpu.sync_copy(acc_shared.at[pl.ds(s * ROWS_PER_TILE, ROWS_PER_TILE)],
                    out_hbm.at[pl.ds(lo + s * ROWS_PER_TILE, ROWS_PER_TILE)])

    @pl.when(s == NS - 1)
    def _write_tail():
        pltpu.sync_copy(acc_shared.at[pl.ds(NS * ROWS_PER_TILE, TAIL_ROWS)],
                        out_hbm.at[pl.ds(lo + NS * ROWS_PER_TILE, TAIL_ROWS)])


def _spmm(support, col, row, ew, b):
    kern = functools.partial(
        pl.kernel,
        mesh=plsc.VectorSubcoreMesh(core_axis_name="c", subcore_axis_name="s"),
        out_type=jax.ShapeDtypeStruct((N_NODES, D), jnp.float32),
        scratch_types=[
            pltpu.VMEM_SHARED((HALF_NODES, D), jnp.float32),
            pltpu.VMEM((CH,), jnp.int32),
            pltpu.VMEM((CH,), jnp.int32),
            pltpu.VMEM((CH,), jnp.float32),
            pltpu.VMEM((CH, D), jnp.float32),
            pltpu.VMEM((D,), jnp.float32),
            pltpu.VMEM((ROWS_PER_TILE, D), jnp.float32),
            pltpu.SemaphoreType.DMA,
            pltpu.SemaphoreType.DMA,
        ],
    )(_spmm_body)
    return kern(support, col, row, ew, b)


def kernel(x, edge_index, edge_weight, W, b):
    ei = edge_index.astype(jnp.int32)
    pad = E_PAD - N_EDGES
    row = jnp.concatenate([ei[0], jnp.zeros((pad,), jnp.int32)])
    col = jnp.concatenate([ei[1], jnp.zeros((pad,), jnp.int32)])
    ew = jnp.concatenate([edge_weight.astype(jnp.float32),
                          jnp.zeros((pad,), jnp.float32)])

    support = _matmul(x, W)
    return _spmm(support, col, row, ew, b)


# speedup vs baseline: 1.0524x; 1.0524x over previous
import functools

import jax
import jax.numpy as jnp
from jax import lax
from jax.experimental import pallas as pl
from jax.experimental.pallas import tpu as pltpu
from jax.experimental.pallas import tpu_sc as plsc

N_NODES = 10000
N_EDGES = 320000
D = 128

NC = 2
NS = 16
HALF_NODES = N_NODES // NC
CH = 128
CHUNKS_PER_TILE = 157
E_PAD = NS * CHUNKS_PER_TILE * CH
ROWS_PER_TILE = 312
TAIL_ROWS = HALF_NODES - NS * ROWS_PER_TILE


def _matmul_body(x_ref, w_ref, o_ref):
    o_ref[...] = jnp.dot(x_ref[...], w_ref[...],
                         preferred_element_type=jnp.float32)


def _matmul(x, W):
    blk = 2000
    return pl.pallas_call(
        _matmul_body,
        grid=(N_NODES // blk,),
        in_specs=[
            pl.BlockSpec((blk, D), lambda i: (i, 0)),
            pl.BlockSpec((D, D), lambda i: (0, 0)),
        ],
        out_specs=pl.BlockSpec((blk, D), lambda i: (i, 0)),
        out_shape=jax.ShapeDtypeStruct((N_NODES, D), jnp.float32),
    )(x, W)


def _spmm_body(support_hbm, col_hbm, row_hbm, ew_hbm, b_hbm, out_hbm,
               acc_shared, colbuf, rowidx, wbuf, rowsbuf, bbuf, zbuf, sem,
               sem_s):
    c = lax.axis_index("c")
    s = lax.axis_index("s")
    lo = c * HALF_NODES

    pltpu.sync_copy(b_hbm, bbuf)

    def _fill_body(r, _):
        for j in range(D // 16):
            sl = pl.ds(j * 16, 16)
            zbuf[r, sl] = bbuf[sl]
        return 0
    lax.fori_loop(0, ROWS_PER_TILE, _fill_body, 0)
    pltpu.sync_copy(zbuf, acc_shared.at[pl.ds(s * ROWS_PER_TILE, ROWS_PER_TILE)])

    @pl.when(s == NS - 1)
    def _fill_tail():
        pltpu.sync_copy(zbuf.at[pl.ds(0, TAIL_ROWS)],
                        acc_shared.at[pl.ds(NS * ROWS_PER_TILE, TAIL_ROWS)])
    plsc.subcore_barrier()

    zv = jnp.zeros((16,), jnp.float32)

    def _zrb_body(e, _):
        for j in range(D // 16):
            rowsbuf[e, pl.ds(j * 16, 16)] = zv
        return 0
    lax.fori_loop(0, CH, _zrb_body, 0)
    zidx = jnp.zeros((16,), jnp.int32)
    for g in range(CH // 16):
        rowidx[pl.ds(g * 16, 16)] = zidx
    pltpu.async_copy(rowsbuf, acc_shared.at[rowidx], sem_s, add=True)

    def _chunk_body(k, _):
        base = (s * CHUNKS_PER_TILE + k) * CH
        pltpu.sync_copy(col_hbm.at[pl.ds(base, CH)], colbuf)
        pltpu.make_async_copy(rowsbuf, acc_shared.at[rowidx], sem_s).wait()
        gather = pltpu.async_copy(support_hbm.at[colbuf], rowsbuf, sem)
        pltpu.sync_copy(ew_hbm.at[pl.ds(base, CH)], wbuf)
        pltpu.sync_copy(row_hbm.at[pl.ds(base, CH)], rowidx)
        gather.wait()

        def _scale_body(g, _):
            sl16 = pl.ds(g * 16, 16)
            rl = rowidx[sl16] - lo
            m = (rl >= 0) & (rl < HALF_NODES)
            wsel = jnp.where(m, wbuf[sl16], 0.0)
            rowidx[sl16] = jnp.where(m, rl, 0)
            for e2 in range(16):
                w = wsel[e2]
                e = g * 16 + e2
                for j in range(D // 16):
                    sl = pl.ds(j * 16, 16)
                    rowsbuf[e, sl] = rowsbuf[e, sl] * w
            return 0
        lax.fori_loop(0, CH // 16, _scale_body, 0)

        pltpu.async_copy(rowsbuf, acc_shared.at[rowidx], sem_s, add=True)
        return 0
    lax.fori_loop(0, CHUNKS_PER_TILE, _chunk_body, 0)
    pltpu.make_async_copy(rowsbuf, acc_shared.at[rowidx], sem_s).wait()

    plsc.subcore_barrier()
    pltpu.sync_copy(acc_shared.at[pl.ds(s * ROWS_PER_TILE, ROWS_PER_TILE)],
                    out_hbm.at[pl.ds(lo + s * ROWS_PER_TILE, ROWS_PER_TILE)])

    @pl.when(s == NS - 1)
    def _write_tail():
        pltpu.sync_copy(acc_shared.at[pl.ds(NS * ROWS_PER_TILE, TAIL_ROWS)],
                        out_hbm.at[pl.ds(lo + NS * ROWS_PER_TILE, TAIL_ROWS)])


def _spmm(support, col, row, ew, b):
    kern = functools.partial(
        pl.kernel,
        mesh=plsc.VectorSubcoreMesh(core_axis_name="c", subcore_axis_name="s"),
        out_type=jax.ShapeDtypeStruct((N_NODES, D), jnp.float32),
        scratch_types=[
            pltpu.VMEM_SHARED((HALF_NODES, D), jnp.float32),
            pltpu.VMEM((CH,), jnp.int32),
            pltpu.VMEM((CH,), jnp.int32),
            pltpu.VMEM((CH,), jnp.float32),
            pltpu.VMEM((CH, D), jnp.float32),
            pltpu.VMEM((D,), jnp.float32),
            pltpu.VMEM((ROWS_PER_TILE, D), jnp.float32),
            pltpu.SemaphoreType.DMA,
            pltpu.SemaphoreType.DMA,
        ],
    )(_spmm_body)
    return kern(support, col, row, ew, b)


def kernel(x, edge_index, edge_weight, W, b):
    ei = edge_index.astype(jnp.int32)
    pad = E_PAD - N_EDGES
    row = jnp.concatenate([ei[0], jnp.zeros((pad,), jnp.int32)])
    col = jnp.concatenate([ei[1], jnp.zeros((pad,), jnp.int32)])
    ew = jnp.concatenate([edge_weight.astype(jnp.float32),
                          jnp.zeros((pad,), jnp.float32)])

    support = _matmul(x, W)
    return _spmm(support, col, row, ew, b)
